# Initial kernel scaffold; baseline (speedup 1.0000x reference)
import jax
import jax.numpy as jnp
from jax.experimental import pallas as pl


def kernel(tf_queries, queries, keys, values, mask, W, b):
    raise NotImplementedError("write your pallas kernel here")


# speedup vs baseline: 34.0112x; 34.0112x over previous
import functools
import numpy as np

import jax
import jax.numpy as jnp
from jax import lax
from jax.experimental import pallas as pl
from jax.experimental.pallas import tpu as pltpu

B, L, H, E = 1, 2048, 12, 64
SCALE = 1.0 / np.sqrt(64)
NTOP = min(int(5 * np.ceil(np.log(L))), L)
F = L // 2 + 1
FP = 1152
HG = 4


def _dft_mats():
    f = np.arange(FP)
    s = np.arange(L)
    ang = 2.0 * np.pi / L * np.outer(s, f)
    valid = (f < F).astype(np.float32)
    cos_f = (np.cos(ang) * valid).astype(np.float32)
    sin_f = (np.sin(ang) * valid).astype(np.float32)
    w = np.where((f == 0) | (f == L // 2), 1.0, 2.0) * valid
    ang_i = 2.0 * np.pi / L * np.outer(f, s)
    icos = (np.cos(ang_i) * w[:, None]).astype(np.float32)
    isin = (np.sin(ang_i) * w[:, None]).astype(np.float32)
    return cos_f, sin_f, icos, isin


_COS, _SIN, _ICOS, _ISIN = _dft_mats()


FB = 384
NFB = FP // FB


def _amp_body(qt, kt, tfqt, cos, sin, icos, isin, amp_t_ref, amp_tf_ref,
              c_scr):
    dot = functools.partial(jnp.dot, preferred_element_type=jnp.float32,
                            precision=lax.Precision.HIGHEST)
    fb = pl.program_id(1)
    q = qt[...]
    k = kt[...]
    aq = dot(q, cos[...])
    bq = dot(q, sin[...])
    ak = dot(k, cos[...])
    bk = dot(k, sin[...])
    pr = aq * ak + bq * bk
    pi = aq * bk - bq * ak
    part = dot(pr, icos[...]) - dot(pi, isin[...])

    @pl.when(fb == 0)
    def _():
        c_scr[...] = part

    @pl.when(fb != 0)
    def _():
        c_scr[...] += part

    @pl.when(fb == NFB - 1)
    def _():
        c = c_scr[...]
        tf = tfqt[...]
        for i in range(HG):
            cs = c[i * E:(i + 1) * E, :]
            amp_t_ref[0, i, :] = jnp.sum(cs * cs, axis=0)
            ts = tf[i * E:(i + 1) * E, :]
            amp_tf_ref[0, i, :] = jnp.sum(ts * ts, axis=0)


def _amplitudes(qt_all, kt_all, tfqt_all):
    grid = H // HG
    blk = pl.BlockSpec((HG * E, L), lambda g, fb: (g, 0))
    a_t, a_tf = pl.pallas_call(
        _amp_body,
        grid=(grid, NFB),
        in_specs=[blk, blk, blk,
                  pl.BlockSpec((L, FB), lambda g, fb: (0, fb)),
                  pl.BlockSpec((L, FB), lambda g, fb: (0, fb)),
                  pl.BlockSpec((FB, L), lambda g, fb: (fb, 0)),
                  pl.BlockSpec((FB, L), lambda g, fb: (fb, 0))],
        out_specs=[pl.BlockSpec((1, HG, L), lambda g, fb: (g, 0, 0)),
                   pl.BlockSpec((1, HG, L), lambda g, fb: (g, 0, 0))],
        out_shape=[jax.ShapeDtypeStruct((grid, HG, L), jnp.float32),
                   jax.ShapeDtypeStruct((grid, HG, L), jnp.float32)],
        scratch_shapes=[pltpu.VMEM((HG * E, L), jnp.float32)],
    )(qt_all, kt_all, tfqt_all, _COS, _SIN, _ICOS, _ISIN)
    return a_t.reshape(H, L), a_tf.reshape(H, L)


def _attn_body(qsel, tfqsel, kt, tfqt, v, wsel, b, o_ref):
    dot = functools.partial(jnp.dot, preferred_element_type=jnp.float32)
    s_t = dot(qsel[0], kt[0]) * SCALE
    s_tf = dot(tfqsel[0], tfqt[0]) * SCALE
    s2 = jnp.concatenate([s_t, s_tf], axis=0)
    m = jnp.max(s2, axis=0, keepdims=True)
    ez = jnp.exp(s2 - m)
    p = ez / jnp.sum(ez, axis=0, keepdims=True)
    wv = dot(wsel[0], v[0])
    o = lax.dot_general(p, wv, (((0,), (0,)), ((), ())),
                        preferred_element_type=jnp.float32)
    bv = dot(b[...], v[0])
    o_ref[0] = o + bv


def _attention(qsel, tfqsel, kt_h, tfqt_h, v_h, wsel, b2):
    sel = pl.BlockSpec((1, NTOP, E), lambda h: (h, 0, 0))
    return pl.pallas_call(
        _attn_body,
        grid=(H,),
        in_specs=[sel, sel,
                  pl.BlockSpec((1, E, L), lambda h: (h, 0, 0)),
                  pl.BlockSpec((1, E, L), lambda h: (h, 0, 0)),
                  pl.BlockSpec((1, L, E), lambda h: (h, 0, 0)),
                  pl.BlockSpec((1, 2 * NTOP, L), lambda h: (h, 0, 0)),
                  pl.BlockSpec((1, L), lambda h: (0, 0))],
        out_specs=pl.BlockSpec((1, L, E), lambda h: (h, 0, 0)),
        out_shape=jax.ShapeDtypeStruct((H, L, E), jnp.float32),
    )(qsel, tfqsel, kt_h, tfqt_h, v_h, wsel, b2)


def kernel(tf_queries, queries, keys, values, mask, W, b):
    del mask
    qt = queries[0].transpose(1, 2, 0).reshape(H * E, L)
    kt = keys[0].transpose(1, 2, 0).reshape(H * E, L)
    tfqt = tf_queries[0].transpose(1, 2, 0).reshape(H * E, L)

    amp2_t, amp2_tf = _amplitudes(qt, kt, tfqt)

    _, idx_t = lax.top_k(amp2_t, NTOP)
    _, idx_tf = lax.top_k(amp2_tf, NTOP)

    q_heads = queries[0].transpose(1, 0, 2)
    tfq_heads = tf_queries[0].transpose(1, 0, 2)
    qsel = jnp.take_along_axis(q_heads, idx_t[..., None], axis=1)
    tfqsel = jnp.take_along_axis(tfq_heads, idx_tf[..., None], axis=1)

    wt = W.T
    cidx = jnp.concatenate([idx_t, idx_tf + L], axis=1)
    wsel = jnp.take(wt, cidx, axis=0)

    kt_h = kt.reshape(H, E, L)
    tfqt_h = tfqt.reshape(H, E, L)
    v_h = values[0].transpose(1, 0, 2)

    o = _attention(qsel, tfqsel, kt_h, tfqt_h, v_h, wsel, b[None, :])
    return o.transpose(1, 0, 2)[None]
